# baseline (speedup 1.0000x reference)
import functools

import jax
import jax.numpy as jnp
from jax import lax
from jax.experimental import pallas as pl
from jax.experimental.pallas import tpu as pltpu
from jax.experimental.pallas import tpu_sc as plsc

_L = 16
_C = 10
_CB = _C * _L


def _make_sc_sampler(nrows, n, ns):
    nb = n // _CB
    nbv = (nb + _L - 1) // _L
    nbp = nbv * _L
    mesh = plsc.VectorSubcoreMesh(core_axis_name="c", subcore_axis_name="s")
    nw = 32

    @functools.partial(
        pl.kernel,
        mesh=mesh,
        out_type=jax.ShapeDtypeStruct((nrows, _L), jnp.int32),
        scratch_types=[
            pltpu.VMEM((n,), jnp.float32),
            pltpu.VMEM((_L,), jnp.float32),
            pltpu.SMEM((nbp,), jnp.float32),
            pltpu.VMEM((_L,), jnp.int32),
        ],
        compiler_params=pltpu.CompilerParams(needs_layout_passes=False),
    )
    def sampler(x_hbm, u_hbm, out_hbm, row_v, u_v, coarse_s, res_v):
        wid = lax.axis_index("s") * 2 + lax.axis_index("c")
        pltpu.sync_copy(x_hbm.at[wid], row_v)
        pltpu.sync_copy(u_hbm.at[wid], u_v)

        def blk(b, total):
            acc = jnp.zeros((_L,), jnp.float32)
            base = b * _CB
            for c in range(_C):
                acc = acc + row_v[pl.ds(base + c * _L, _L)]
            total = total + jnp.sum(acc)
            coarse_s[b] = total
            return total

        total = lax.fori_loop(0, nb, blk, jnp.float32(0.0), unroll=4)

        iota = lax.iota(jnp.int32, _L)
        u_vec = u_v[...]

        def sample(k, res):
            u_k = jnp.sum(jnp.where(iota == k, u_vec, 0.0))
            t_k = u_k * total
            tks = jnp.full((_L,), t_k, jnp.float32)

            def bstep(_, lohi):
                lo, hi = lohi
                mid = lax.shift_right_logical(lo + hi, 1)
                val = coarse_s[jnp.minimum(mid, nb - 1)]
                active = lo < hi
                go = jnp.logical_and(val <= t_k, active)
                lo2 = jnp.where(go, mid + 1, lo)
                hi2 = jnp.where(go, hi, jnp.where(active, mid, hi))
                return lo2, hi2

            nfull, _ = lax.fori_loop(
                0, nb.bit_length(), bstep, (jnp.int32(0), jnp.int32(nb)))
            pe = jnp.where(
                nfull > 0, coarse_s[jnp.maximum(nfull - 1, 0)], 0.0)
            bb = jnp.minimum(nfull, nb - 1)
            fbase = bb * _CB

            def fine(c, st):
                carry, cntv = st
                v = row_v[pl.ds(fbase + c * _L, _L)]
                pref = plsc.cumsum(v) + carry
                cntv = cntv + jnp.where(pref <= tks, 1, 0)
                return carry + jnp.sum(v), cntv

            st = (pe, jnp.zeros((_L,), jnp.int32))
            for c in range(_C):
                st = fine(c, st)
            _, cntv = st
            idx_k = jnp.minimum(nfull * _CB + jnp.sum(cntv), n - 1)
            return jnp.where(iota == k, idx_k, res)

        res = jnp.zeros((_L,), jnp.int32)
        for k in range(ns):
            res = sample(k, res)
        res_v[...] = res
        pltpu.sync_copy(res_v, out_hbm.at[wid])

    return sampler


@functools.lru_cache(maxsize=None)
def _sampler_cached(nrows, n, ns):
    return _make_sc_sampler(nrows, n, ns)


def kernel(x, num_samples, replacement, out):
    del replacement
    nrows, n = x.shape
    ns = out.shape[-1]
    u = jax.random.uniform(jax.random.key(42), (nrows, ns), dtype=x.dtype)
    u16 = jnp.concatenate(
        [u, jnp.ones((nrows, _L - ns), dtype=u.dtype)], axis=1)
    res = _sampler_cached(nrows, n, ns)(x, u16)
    idx = res[:, :ns]
    idx = idx + (jnp.asarray(num_samples, dtype=idx.dtype) - ns)
    return idx.astype(out.dtype)


# speedup vs baseline: 1.0048x; 1.0048x over previous
import functools

import jax
import jax.numpy as jnp
from jax import lax
from jax.experimental import pallas as pl
from jax.experimental.pallas import tpu as pltpu
from jax.experimental.pallas import tpu_sc as plsc

_L = 16
_C = 10
_CB = _C * _L


def _make_sc_sampler(nrows, n, ns):
    nb = n // _CB
    nbv = (nb + _L - 1) // _L
    nbp = nbv * _L
    mesh = plsc.VectorSubcoreMesh(core_axis_name="c", subcore_axis_name="s")
    nw = 32

    @functools.partial(
        pl.kernel,
        mesh=mesh,
        out_type=jax.ShapeDtypeStruct((nrows, _L), jnp.int32),
        scratch_types=[
            pltpu.VMEM((n,), jnp.float32),
            pltpu.VMEM((_L,), jnp.float32),
            pltpu.SMEM((nbp,), jnp.float32),
            pltpu.VMEM((_L,), jnp.int32),
        ],
        compiler_params=pltpu.CompilerParams(needs_layout_passes=False),
    )
    def sampler(x_hbm, u_hbm, out_hbm, row_v, u_v, coarse_s, res_v):
        wid = lax.axis_index("s") * 2 + lax.axis_index("c")
        pltpu.sync_copy(x_hbm.at[wid], row_v)
        pltpu.sync_copy(u_hbm.at[wid], u_v)

        def blk(b, total):
            acc = jnp.zeros((_L,), jnp.float32)
            base = b * _CB
            for c in range(_C):
                acc = acc + row_v[pl.ds(base + c * _L, _L)]
            total = total + jnp.sum(acc)
            coarse_s[b] = total
            return total

        total = lax.fori_loop(0, nb, blk, jnp.float32(0.0), unroll=2)

        iota = lax.iota(jnp.int32, _L)
        u_vec = u_v[...]

        def sample(k, res):
            u_k = jnp.sum(jnp.where(iota == k, u_vec, 0.0))
            t_k = u_k * total
            tks = jnp.full((_L,), t_k, jnp.float32)

            def bstep(_, lohi):
                lo, hi = lohi
                mid = lax.shift_right_logical(lo + hi, 1)
                val = coarse_s[jnp.minimum(mid, nb - 1)]
                active = lo < hi
                go = jnp.logical_and(val <= t_k, active)
                lo2 = jnp.where(go, mid + 1, lo)
                hi2 = jnp.where(go, hi, jnp.where(active, mid, hi))
                return lo2, hi2

            nfull, _ = lax.fori_loop(
                0, nb.bit_length(), bstep, (jnp.int32(0), jnp.int32(nb)))
            pe = jnp.where(
                nfull > 0, coarse_s[jnp.maximum(nfull - 1, 0)], 0.0)
            bb = jnp.minimum(nfull, nb - 1)
            fbase = bb * _CB

            def fine(c, st):
                carry, cntv = st
                v = row_v[pl.ds(fbase + c * _L, _L)]
                pref = plsc.cumsum(v) + carry
                cntv = cntv + jnp.where(pref <= tks, 1, 0)
                return carry + jnp.sum(v), cntv

            st = (pe, jnp.zeros((_L,), jnp.int32))
            for c in range(_C):
                st = fine(c, st)
            _, cntv = st
            idx_k = jnp.minimum(nfull * _CB + jnp.sum(cntv), n - 1)
            return jnp.where(iota == k, idx_k, res)

        res = jnp.zeros((_L,), jnp.int32)
        for k in range(ns):
            res = sample(k, res)
        res_v[...] = res
        pltpu.sync_copy(res_v, out_hbm.at[wid])

    return sampler


@functools.lru_cache(maxsize=None)
def _sampler_cached(nrows, n, ns):
    return _make_sc_sampler(nrows, n, ns)


def kernel(x, num_samples, replacement, out):
    del replacement
    nrows, n = x.shape
    ns = out.shape[-1]
    u = jax.random.uniform(jax.random.key(42), (nrows, ns), dtype=x.dtype)
    u16 = jnp.concatenate(
        [u, jnp.ones((nrows, _L - ns), dtype=u.dtype)], axis=1)
    res = _sampler_cached(nrows, n, ns)(x, u16)
    idx = res[:, :ns]
    idx = idx + (jnp.asarray(num_samples, dtype=idx.dtype) - ns)
    return idx.astype(out.dtype)
